# Initial kernel scaffold; baseline (speedup 1.0000x reference)
import jax
import jax.numpy as jnp
from jax.experimental import pallas as pl


def kernel(x, edge_index, W1, b1, g1, be1, W2, b2, g2, be2, Wf1, bf1, Wf2, bf2):
    raise NotImplementedError("write your pallas kernel here")


# speedup vs baseline: 8.3226x; 8.3226x over previous
import functools

import jax
import jax.numpy as jnp
from jax import lax
from jax.experimental import pallas as pl
from jax.experimental.pallas import tpu as pltpu
from jax.experimental.pallas import tpu_sc as plsc

_N = 10000
_E = 320000
_D = 128
_C = 64
_NC = 2
_NS = 16
_NW = _NC * _NS
_CH = 128
_NCHUNK = 80
_EPW = _CH * _NCHUNK
_EPAD = _EPW * _NW
_ROWS2D = _EPAD // _CH
_NP = 10240
_RPT = _NP // _NS


def _deg_body(dst2, out, dst_v, dcur_v, ones_v, zer_v, acc_sh):
    c = lax.axis_index("c")
    s = lax.axis_index("s")
    w = c * _NS + s
    one = jnp.ones((16,), jnp.float32)
    zero = jnp.zeros((16,), jnp.float32)

    def fill_ones(i, _):
        ones_v[i, :] = one
        return 0

    lax.fori_loop(0, _CH, fill_ones, 0)

    def fill_zeros(i, _):
        zer_v[i, :] = zero
        return 0

    lax.fori_loop(0, _RPT, fill_zeros, 0)
    pltpu.sync_copy(zer_v, acc_sh.at[pl.ds(s * _RPT, _RPT)])
    pltpu.sync_copy(dst2.at[pl.ds(w * _NCHUNK, _NCHUNK)], dst_v)
    plsc.subcore_barrier()

    def body(j, _):
        def cpidx(k, _2):
            dcur_v[pl.ds(k * 16, 16)] = dst_v[j, pl.ds(k * 16, 16)]
            return 0

        lax.fori_loop(0, _CH // 16, cpidx, 0)
        pltpu.sync_copy(ones_v, acc_sh.at[dcur_v], add=True)
        return 0

    lax.fori_loop(0, _NCHUNK, body, 0)
    plsc.subcore_barrier()
    ob = s * _RPT
    pltpu.sync_copy(acc_sh.at[pl.ds(ob, _RPT)], zer_v)
    pltpu.sync_copy(zer_v, out.at[c, pl.ds(ob, _RPT)])


def _agg_body(hp, src2, dst2, out, src_v, dst_v, scur_v, dcur_v, rows_v,
              acc_sh):
    c = lax.axis_index("c")
    s = lax.axis_index("s")
    w = c * _NS + s
    zero = jnp.zeros((16,), jnp.float32)

    def zrow(i, _):
        def zcol(k, _2):
            rows_v[i, pl.ds(k * 16, 16)] = zero
            return 0

        lax.fori_loop(0, _D // 16, zcol, 0)
        return 0

    lax.fori_loop(0, _CH, zrow, 0)
    base = s * _RPT
    for k in range(0, _RPT, _CH):
        pltpu.sync_copy(rows_v, acc_sh.at[pl.ds(base + k, _CH)])
    pltpu.sync_copy(src2.at[pl.ds(w * _NCHUNK, _NCHUNK)], src_v)
    pltpu.sync_copy(dst2.at[pl.ds(w * _NCHUNK, _NCHUNK)], dst_v)
    plsc.subcore_barrier()

    def body(j, _):
        def cpidx(k, _2):
            scur_v[pl.ds(k * 16, 16)] = src_v[j, pl.ds(k * 16, 16)]
            dcur_v[pl.ds(k * 16, 16)] = dst_v[j, pl.ds(k * 16, 16)]
            return 0

        lax.fori_loop(0, _CH // 16, cpidx, 0)
        pltpu.sync_copy(hp.at[scur_v], rows_v)
        pltpu.sync_copy(rows_v, acc_sh.at[dcur_v], add=True)
        return 0

    lax.fori_loop(0, _NCHUNK, body, 0)
    plsc.subcore_barrier()
    ob = s * _RPT
    for k in range(0, _RPT, _CH):
        pltpu.sync_copy(acc_sh.at[pl.ds(ob + k, _CH)], rows_v)
        pltpu.sync_copy(rows_v, out.at[c, pl.ds(ob + k, _CH)])


@functools.cache
def _get_deg_kernel():
    mesh = plsc.VectorSubcoreMesh(
        core_axis_name="c", subcore_axis_name="s",
        num_cores=_NC, num_subcores=_NS)
    return pl.kernel(
        _deg_body,
        out_type=jax.ShapeDtypeStruct((_NC, _NP, 16), jnp.float32),
        mesh=mesh,
        compiler_params=pltpu.CompilerParams(use_tc_tiling_on_sc=False),
        scratch_types=[
            pltpu.VMEM((_NCHUNK, _CH), jnp.int32),
            pltpu.VMEM((_CH,), jnp.int32),
            pltpu.VMEM((_CH, 16), jnp.float32),
            pltpu.VMEM((_RPT, 16), jnp.float32),
            pltpu.VMEM_SHARED((_NP, 16), jnp.float32),
        ],
    )


@functools.cache
def _get_agg_kernel():
    mesh = plsc.VectorSubcoreMesh(
        core_axis_name="c", subcore_axis_name="s",
        num_cores=_NC, num_subcores=_NS)
    return pl.kernel(
        _agg_body,
        out_type=jax.ShapeDtypeStruct((_NC, _NP, _D), jnp.float32),
        mesh=mesh,
        compiler_params=pltpu.CompilerParams(use_tc_tiling_on_sc=False),
        scratch_types=[
            pltpu.VMEM((_NCHUNK, _CH), jnp.int32),
            pltpu.VMEM((_NCHUNK, _CH), jnp.int32),
            pltpu.VMEM((_CH,), jnp.int32),
            pltpu.VMEM((_CH,), jnp.int32),
            pltpu.VMEM((_CH, _D), jnp.float32),
            pltpu.VMEM_SHARED((_NP, _D), jnp.float32),
        ],
    )


def _dinv_from(degp_ref):
    deg = degp_ref[0, 0:_N, 0:1] + degp_ref[1, 0:_N, 0:1] + 1.0
    return lax.rsqrt(jnp.maximum(deg, 1.0))


def _dot(a, b):
    return jnp.dot(a, b, preferred_element_type=jnp.float32,
                   precision=lax.Precision.HIGHEST)


def _tc1_body(x_ref, w1_ref, degp_ref, out_ref):
    dinv = _dinv_from(degp_ref)
    out_ref[...] = dinv * _dot(x_ref[...], w1_ref[...])


def _bn_relu(h, g_ref, be_ref):
    mu = jnp.mean(h, axis=0, keepdims=True)
    var = jnp.mean((h - mu) ** 2, axis=0, keepdims=True)
    return jnp.maximum((h - mu) * lax.rsqrt(var + 1e-5) * g_ref[...]
                       + be_ref[...], 0.0)


def _tc2_body(acc_ref, hp_ref, degp_ref, b_ref, g_ref, be_ref, w2_ref,
              out_ref):
    dinv = _dinv_from(degp_ref)
    h = dinv * (acc_ref[0, 0:_N] + acc_ref[1, 0:_N] + hp_ref[...]) + b_ref[...]
    r = _bn_relu(h, g_ref, be_ref)
    out_ref[...] = dinv * _dot(r, w2_ref[...])


def _tc3_body(acc_ref, hp_ref, degp_ref, b_ref, g_ref, be_ref,
              wf1_ref, bf1_ref, wf2_ref, bf2_ref, out_ref):
    dinv = _dinv_from(degp_ref)
    h = dinv * (acc_ref[0, 0:_N] + acc_ref[1, 0:_N] + hp_ref[...]) + b_ref[...]
    r = _bn_relu(h, g_ref, be_ref)
    m = jnp.maximum(_dot(r, wf1_ref[...]) + bf1_ref[...], 0.0)
    o = _dot(m, wf2_ref[...]) + bf2_ref[...]
    sh = o - jnp.max(o, axis=1, keepdims=True)
    out_ref[...] = sh - jnp.log(jnp.sum(jnp.exp(sh), axis=1, keepdims=True))


_tc1_call = pl.pallas_call(
    _tc1_body, out_shape=jax.ShapeDtypeStruct((_N, _D), jnp.float32))
_tc2_call = pl.pallas_call(
    _tc2_body, out_shape=jax.ShapeDtypeStruct((_N, _D), jnp.float32))
_tc3_call = pl.pallas_call(
    _tc3_body, out_shape=jax.ShapeDtypeStruct((_N, _C), jnp.float32))


def kernel(x, edge_index, W1, b1, g1, be1, W2, b2, g2, be2, Wf1, bf1,
           Wf2, bf2):
    src = edge_index[0].astype(jnp.int32)
    dst = edge_index[1].astype(jnp.int32)
    npad = _EPAD - _E
    src2 = jnp.concatenate(
        [src, jnp.zeros((npad,), jnp.int32)]).reshape(_ROWS2D, _CH)
    dst2 = jnp.concatenate(
        [dst, jnp.full((npad,), _N, jnp.int32)]).reshape(_ROWS2D, _CH)
    degp = _get_deg_kernel()(dst2)
    hp1 = _tc1_call(x, W1, degp)
    acc1 = _get_agg_kernel()(hp1, src2, dst2)
    hp2 = _tc2_call(acc1, hp1, degp, b1.reshape(1, -1), g1.reshape(1, -1),
                    be1.reshape(1, -1), W2)
    acc2 = _get_agg_kernel()(hp2, src2, dst2)
    return _tc3_call(acc2, hp2, degp, b2.reshape(1, -1), g2.reshape(1, -1),
                     be2.reshape(1, -1), Wf1, bf1.reshape(1, -1),
                     Wf2, bf2.reshape(1, -1))


# speedup vs baseline: 9.3287x; 1.1209x over previous
import functools

import jax
import jax.numpy as jnp
from jax import lax
from jax.experimental import pallas as pl
from jax.experimental.pallas import tpu as pltpu
from jax.experimental.pallas import tpu_sc as plsc

_N = 10000
_E = 320000
_D = 128
_C = 64
_NC = 2
_NS = 16
_NW = _NC * _NS
_CH = 128
_NCHUNK = 80
_EPW = _CH * _NCHUNK
_EPAD = _EPW * _NW
_ROWS2D = _EPAD // _CH
_NP = 10240
_RPT = _NP // _NS


def _deg_body(dst2, out, dst_v, dcur_v, ones_v, zer_v, acc_sh):
    c = lax.axis_index("c")
    s = lax.axis_index("s")
    w = c * _NS + s
    one = jnp.ones((16,), jnp.float32)
    zero = jnp.zeros((16,), jnp.float32)

    def fill_ones(i, _):
        ones_v[i, :] = one
        return 0

    lax.fori_loop(0, _CH, fill_ones, 0)

    def fill_zeros(i, _):
        zer_v[i, :] = zero
        return 0

    lax.fori_loop(0, _RPT, fill_zeros, 0)
    pltpu.sync_copy(zer_v, acc_sh.at[pl.ds(s * _RPT, _RPT)])
    pltpu.sync_copy(dst2.at[pl.ds(w * _NCHUNK, _NCHUNK)], dst_v)
    plsc.subcore_barrier()

    def body(j, _):
        def cpidx(k, _2):
            dcur_v[pl.ds(k * 16, 16)] = dst_v[j, pl.ds(k * 16, 16)]
            return 0

        lax.fori_loop(0, _CH // 16, cpidx, 0)
        pltpu.sync_copy(ones_v, acc_sh.at[dcur_v], add=True)
        return 0

    lax.fori_loop(0, _NCHUNK, body, 0)
    plsc.subcore_barrier()
    ob = s * _RPT
    pltpu.sync_copy(acc_sh.at[pl.ds(ob, _RPT)], zer_v)
    pltpu.sync_copy(zer_v, out.at[c, pl.ds(ob, _RPT)])


_NOUT = _NCHUNK // 2


def _agg_body(hp, src2, dst2, out, sa0, da0, sa1, da1, rb0, rb1,
              acc_sh, gsem0, gsem1, isem0, isem1):
    c = lax.axis_index("c")
    s = lax.axis_index("s")
    w = c * _NS + s
    base_row = w * _NCHUNK
    zero = jnp.zeros((16,), jnp.float32)

    def zrow(i, _):
        def zcol(k, _2):
            rb0[i, pl.ds(k * 16, 16)] = zero
            return 0

        lax.fori_loop(0, _D // 16, zcol, 0)
        return 0

    lax.fori_loop(0, _CH, zrow, 0)
    base = s * _RPT
    for k in range(0, _RPT, _CH):
        pltpu.sync_copy(rb0, acc_sh.at[pl.ds(base + k, _CH)])
    pltpu.sync_copy(src2.at[base_row], sa0)
    pltpu.sync_copy(dst2.at[base_row], da0)
    pltpu.async_copy(hp.at[sa0], rb0, gsem0)
    pltpu.async_copy(src2.at[base_row + 1], sa1, isem1)
    pltpu.async_copy(dst2.at[base_row + 1], da1, isem1)
    plsc.subcore_barrier()

    def outer(g, _):
        pltpu.make_async_copy(src2.at[base_row], sa1, isem1).wait()
        pltpu.make_async_copy(dst2.at[base_row], da1, isem1).wait()
        pltpu.async_copy(hp.at[sa1], rb1, gsem1)
        pltpu.make_async_copy(hp.at[pl.ds(0, _CH)], rb0, gsem0).wait()
        pltpu.sync_copy(rb0, acc_sh.at[da0], add=True)

        @pl.when(g < _NOUT - 1)
        def _():
            d1 = pltpu.async_copy(src2.at[base_row + 2 * g + 2], sa0, isem0)
            d2 = pltpu.async_copy(dst2.at[base_row + 2 * g + 2], da0, isem0)
            d1.wait()
            d2.wait()
            pltpu.async_copy(hp.at[sa0], rb0, gsem0)

        pltpu.make_async_copy(hp.at[pl.ds(0, _CH)], rb1, gsem1).wait()
        pltpu.sync_copy(rb1, acc_sh.at[da1], add=True)

        @pl.when(g < _NOUT - 1)
        def _():
            pltpu.async_copy(src2.at[base_row + 2 * g + 3], sa1, isem1)
            pltpu.async_copy(dst2.at[base_row + 2 * g + 3], da1, isem1)

        return 0

    lax.fori_loop(0, _NOUT, outer, 0)
    plsc.subcore_barrier()
    ob = s * _RPT
    for k in range(0, _RPT, _CH):
        pltpu.sync_copy(acc_sh.at[pl.ds(ob + k, _CH)], rb0)
        pltpu.sync_copy(rb0, out.at[c, pl.ds(ob + k, _CH)])


@functools.cache
def _get_deg_kernel():
    mesh = plsc.VectorSubcoreMesh(
        core_axis_name="c", subcore_axis_name="s",
        num_cores=_NC, num_subcores=_NS)
    return pl.kernel(
        _deg_body,
        out_type=jax.ShapeDtypeStruct((_NC, _NP, 16), jnp.float32),
        mesh=mesh,
        compiler_params=pltpu.CompilerParams(use_tc_tiling_on_sc=False),
        scratch_types=[
            pltpu.VMEM((_NCHUNK, _CH), jnp.int32),
            pltpu.VMEM((_CH,), jnp.int32),
            pltpu.VMEM((_CH, 16), jnp.float32),
            pltpu.VMEM((_RPT, 16), jnp.float32),
            pltpu.VMEM_SHARED((_NP, 16), jnp.float32),
        ],
    )


@functools.cache
def _get_agg_kernel():
    mesh = plsc.VectorSubcoreMesh(
        core_axis_name="c", subcore_axis_name="s",
        num_cores=_NC, num_subcores=_NS)
    return pl.kernel(
        _agg_body,
        out_type=jax.ShapeDtypeStruct((_NC, _NP, _D), jnp.float32),
        mesh=mesh,
        compiler_params=pltpu.CompilerParams(use_tc_tiling_on_sc=False),
        scratch_types=(
            [pltpu.VMEM((_CH,), jnp.int32)] * 4
            + [pltpu.VMEM((_CH, _D), jnp.float32)] * 2
            + [pltpu.VMEM_SHARED((_NP, _D), jnp.float32)]
            + [pltpu.SemaphoreType.DMA] * 4
        ),
    )


def _dinv_from(degp_ref):
    deg = degp_ref[0, 0:_N, 0:1] + degp_ref[1, 0:_N, 0:1] + 1.0
    return lax.rsqrt(jnp.maximum(deg, 1.0))


def _dot(a, b):
    return jnp.dot(a, b, preferred_element_type=jnp.float32,
                   precision=lax.Precision.HIGHEST)


def _tc1_body(x_ref, w1_ref, degp_ref, out_ref):
    dinv = _dinv_from(degp_ref)
    out_ref[...] = dinv * _dot(x_ref[...], w1_ref[...])


def _bn_relu(h, g_ref, be_ref):
    mu = jnp.mean(h, axis=0, keepdims=True)
    var = jnp.mean((h - mu) ** 2, axis=0, keepdims=True)
    return jnp.maximum((h - mu) * lax.rsqrt(var + 1e-5) * g_ref[...]
                       + be_ref[...], 0.0)


def _tc2_body(acc_ref, hp_ref, degp_ref, b_ref, g_ref, be_ref, w2_ref,
              out_ref):
    dinv = _dinv_from(degp_ref)
    h = dinv * (acc_ref[0, 0:_N] + acc_ref[1, 0:_N] + hp_ref[...]) + b_ref[...]
    r = _bn_relu(h, g_ref, be_ref)
    out_ref[...] = dinv * _dot(r, w2_ref[...])


def _tc3_body(acc_ref, hp_ref, degp_ref, b_ref, g_ref, be_ref,
              wf1_ref, bf1_ref, wf2_ref, bf2_ref, out_ref):
    dinv = _dinv_from(degp_ref)
    h = dinv * (acc_ref[0, 0:_N] + acc_ref[1, 0:_N] + hp_ref[...]) + b_ref[...]
    r = _bn_relu(h, g_ref, be_ref)
    m = jnp.maximum(_dot(r, wf1_ref[...]) + bf1_ref[...], 0.0)
    o = _dot(m, wf2_ref[...]) + bf2_ref[...]
    sh = o - jnp.max(o, axis=1, keepdims=True)
    out_ref[...] = sh - jnp.log(jnp.sum(jnp.exp(sh), axis=1, keepdims=True))


_tc1_call = pl.pallas_call(
    _tc1_body, out_shape=jax.ShapeDtypeStruct((_N, _D), jnp.float32))
_tc2_call = pl.pallas_call(
    _tc2_body, out_shape=jax.ShapeDtypeStruct((_N, _D), jnp.float32))
_tc3_call = pl.pallas_call(
    _tc3_body, out_shape=jax.ShapeDtypeStruct((_N, _C), jnp.float32))


def kernel(x, edge_index, W1, b1, g1, be1, W2, b2, g2, be2, Wf1, bf1,
           Wf2, bf2):
    src = edge_index[0].astype(jnp.int32)
    dst = edge_index[1].astype(jnp.int32)
    npad = _EPAD - _E
    src2 = jnp.concatenate(
        [src, jnp.zeros((npad,), jnp.int32)]).reshape(_ROWS2D, _CH)
    dst2 = jnp.concatenate(
        [dst, jnp.full((npad,), _N, jnp.int32)]).reshape(_ROWS2D, _CH)
    degp = _get_deg_kernel()(dst2)
    hp1 = _tc1_call(x, W1, degp)
    acc1 = _get_agg_kernel()(hp1, src2, dst2)
    hp2 = _tc2_call(acc1, hp1, degp, b1.reshape(1, -1), g1.reshape(1, -1),
                    be1.reshape(1, -1), W2)
    acc2 = _get_agg_kernel()(hp2, src2, dst2)
    return _tc3_call(acc2, hp2, degp, b2.reshape(1, -1), g2.reshape(1, -1),
                     be2.reshape(1, -1), Wf1, bf1.reshape(1, -1),
                     Wf2, bf2.reshape(1, -1))


# speedup vs baseline: 9.3365x; 1.0008x over previous
import functools

import jax
import jax.numpy as jnp
from jax import lax
from jax.experimental import pallas as pl
from jax.experimental.pallas import tpu as pltpu
from jax.experimental.pallas import tpu_sc as plsc

_N = 10000
_E = 320000
_D = 128
_C = 64
_NC = 2
_NS = 16
_NW = _NC * _NS
_CH = 128
_NCHUNK = 80
_EPW = _CH * _NCHUNK
_EPAD = _EPW * _NW
_ROWS2D = _EPAD // _CH
_NP = 10240
_RPT = _NP // _NS


def _deg_body(dst2, out, dst_v, dcur_v, ones_v, zer_v, acc_sh):
    c = lax.axis_index("c")
    s = lax.axis_index("s")
    w = c * _NS + s
    one = jnp.ones((16,), jnp.float32)
    zero = jnp.zeros((16,), jnp.float32)

    def fill_ones(i, _):
        ones_v[i, :] = one
        return 0

    lax.fori_loop(0, _CH, fill_ones, 0)

    def fill_zeros(i, _):
        zer_v[i, :] = zero
        return 0

    lax.fori_loop(0, _RPT, fill_zeros, 0)
    pltpu.sync_copy(zer_v, acc_sh.at[pl.ds(s * _RPT, _RPT)])
    pltpu.sync_copy(dst2.at[pl.ds(w * _NCHUNK, _NCHUNK)], dst_v)
    plsc.subcore_barrier()

    def body(j, _):
        def cpidx(k, _2):
            dcur_v[pl.ds(k * 16, 16)] = dst_v[j, pl.ds(k * 16, 16)]
            return 0

        lax.fori_loop(0, _CH // 16, cpidx, 0)
        pltpu.sync_copy(ones_v, acc_sh.at[dcur_v], add=True)
        return 0

    lax.fori_loop(0, _NCHUNK, body, 0)
    plsc.subcore_barrier()
    ob = s * _RPT
    pltpu.sync_copy(acc_sh.at[pl.ds(ob, _RPT)], zer_v)
    pltpu.sync_copy(zer_v, out.at[c, pl.ds(ob, _RPT)])


_NOUT = _NCHUNK // 2


def _agg_body(hp, src2, dst2, out, sa0, da0, sa1, da1, rb0, rb1,
              acc_sh, gsem0, gsem1, isem0, isem1):
    c = lax.axis_index("c")
    s = lax.axis_index("s")
    w = c * _NS + s
    base_row = w * _NCHUNK
    zero = jnp.zeros((16,), jnp.float32)

    def zrow(i, _):
        def zcol(k, _2):
            rb0[i, pl.ds(k * 16, 16)] = zero
            return 0

        lax.fori_loop(0, _D // 16, zcol, 0)
        return 0

    lax.fori_loop(0, _CH, zrow, 0)
    base = s * _RPT
    for k in range(0, _RPT, _CH):
        pltpu.sync_copy(rb0, acc_sh.at[pl.ds(base + k, _CH)])
    pltpu.sync_copy(src2.at[base_row], sa0)
    pltpu.sync_copy(dst2.at[base_row], da0)
    pltpu.async_copy(hp.at[sa0], rb0, gsem0)
    pltpu.async_copy(src2.at[base_row + 1], sa1, isem1)
    pltpu.async_copy(dst2.at[base_row + 1], da1, isem1)
    plsc.subcore_barrier()

    def outer(g, _):
        pltpu.make_async_copy(src2.at[base_row], sa1, isem1).wait()
        pltpu.make_async_copy(dst2.at[base_row], da1, isem1).wait()
        pltpu.async_copy(hp.at[sa1], rb1, gsem1)
        pltpu.make_async_copy(hp.at[pl.ds(0, _CH)], rb0, gsem0).wait()
        pltpu.sync_copy(rb0, acc_sh.at[da0], add=True)

        @pl.when(g < _NOUT - 1)
        def _():
            d1 = pltpu.async_copy(src2.at[base_row + 2 * g + 2], sa0, isem0)
            d2 = pltpu.async_copy(dst2.at[base_row + 2 * g + 2], da0, isem0)
            d1.wait()
            d2.wait()
            pltpu.async_copy(hp.at[sa0], rb0, gsem0)

        pltpu.make_async_copy(hp.at[pl.ds(0, _CH)], rb1, gsem1).wait()
        pltpu.sync_copy(rb1, acc_sh.at[da1], add=True)

        @pl.when(g < _NOUT - 1)
        def _():
            pltpu.async_copy(src2.at[base_row + 2 * g + 3], sa1, isem1)
            pltpu.async_copy(dst2.at[base_row + 2 * g + 3], da1, isem1)

        return 0

    lax.fori_loop(0, _NOUT, outer, 0)
    plsc.subcore_barrier()
    ob = s * _RPT
    for k in range(0, _RPT, _CH):
        pltpu.sync_copy(acc_sh.at[pl.ds(ob + k, _CH)], rb0)
        pltpu.sync_copy(rb0, out.at[c, pl.ds(ob + k, _CH)])


@functools.cache
def _get_deg_kernel():
    mesh = plsc.VectorSubcoreMesh(
        core_axis_name="c", subcore_axis_name="s",
        num_cores=_NC, num_subcores=_NS)
    return pl.kernel(
        _deg_body,
        out_type=jax.ShapeDtypeStruct((_NC, _NP, 16), jnp.float32),
        mesh=mesh,
        compiler_params=pltpu.CompilerParams(use_tc_tiling_on_sc=False),
        scratch_types=[
            pltpu.VMEM((_NCHUNK, _CH), jnp.int32),
            pltpu.VMEM((_CH,), jnp.int32),
            pltpu.VMEM((_CH, 16), jnp.float32),
            pltpu.VMEM((_RPT, 16), jnp.float32),
            pltpu.VMEM_SHARED((_NP, 16), jnp.float32),
        ],
    )


@functools.cache
def _get_agg_kernel():
    mesh = plsc.VectorSubcoreMesh(
        core_axis_name="c", subcore_axis_name="s",
        num_cores=_NC, num_subcores=_NS)
    return pl.kernel(
        _agg_body,
        out_type=jax.ShapeDtypeStruct((_NC, _NP, _D), jnp.float32),
        mesh=mesh,
        compiler_params=pltpu.CompilerParams(use_tc_tiling_on_sc=False),
        scratch_types=(
            [pltpu.VMEM((_CH,), jnp.int32)] * 4
            + [pltpu.VMEM((_CH, _D), jnp.float32)] * 2
            + [pltpu.VMEM_SHARED((_NP, _D), jnp.float32)]
            + [pltpu.SemaphoreType.DMA] * 4
        ),
    )


def _dinv_from(degp_ref):
    deg = degp_ref[0, 0:_N, 0:1] + degp_ref[1, 0:_N, 0:1] + 1.0
    return lax.rsqrt(jnp.maximum(deg, 1.0))


def _dot(a, b):
    return jnp.dot(a, b, preferred_element_type=jnp.float32,
                   precision=lax.Precision.HIGHEST)


def _tc1_body(x_ref, w1_ref, degp_ref, out_ref):
    dinv = _dinv_from(degp_ref)
    out_ref[...] = dinv * _dot(x_ref[...], w1_ref[...])


def _bn_relu(h, g_ref, be_ref):
    mu = jnp.mean(h, axis=0, keepdims=True)
    var = jnp.mean((h - mu) ** 2, axis=0, keepdims=True)
    return jnp.maximum((h - mu) * lax.rsqrt(var + 1e-5) * g_ref[...]
                       + be_ref[...], 0.0)


def _tc2_body(acc_ref, hp_ref, degp_ref, b_ref, g_ref, be_ref, w2_ref,
              out_ref):
    dinv = _dinv_from(degp_ref)
    h = dinv * (acc_ref[0, 0:_N] + acc_ref[1, 0:_N] + hp_ref[...]) + b_ref[...]
    r = _bn_relu(h, g_ref, be_ref)
    out_ref[...] = dinv * _dot(r, w2_ref[...])


def _tc3_body(acc_ref, hp_ref, degp_ref, b_ref, g_ref, be_ref,
              wf1_ref, bf1_ref, wf2_ref, bf2_ref, out_ref):
    dinv = _dinv_from(degp_ref)
    h = dinv * (acc_ref[0, 0:_N] + acc_ref[1, 0:_N] + hp_ref[...]) + b_ref[...]
    r = _bn_relu(h, g_ref, be_ref)
    m = jnp.maximum(_dot(r, wf1_ref[...]) + bf1_ref[...], 0.0)
    o = _dot(m, wf2_ref[...]) + bf2_ref[...]
    sh = o - jnp.max(o, axis=1, keepdims=True)
    out_ref[...] = sh - jnp.log(jnp.sum(jnp.exp(sh), axis=1, keepdims=True))


_tc1_call = pl.pallas_call(
    _tc1_body, out_shape=jax.ShapeDtypeStruct((_N, _D), jnp.float32))
_tc2_call = pl.pallas_call(
    _tc2_body, out_shape=jax.ShapeDtypeStruct((_N, _D), jnp.float32))
_tc3_call = pl.pallas_call(
    _tc3_body, out_shape=jax.ShapeDtypeStruct((_N, _C), jnp.float32))


def kernel(x, edge_index, W1, b1, g1, be1, W2, b2, g2, be2, Wf1, bf1,
           Wf2, bf2):
    src = edge_index[0].astype(jnp.int32)
    dst = edge_index[1].astype(jnp.int32)
    npad = _EPAD - _E
    src2 = jnp.concatenate(
        [src, jnp.zeros((npad,), jnp.int32)]).reshape(_ROWS2D, _CH)
    pad_dst = _N + jnp.arange(npad, dtype=jnp.int32) % (_NP - _N)
    dst2 = jnp.concatenate([dst, pad_dst]).reshape(_ROWS2D, _CH)
    degp = _get_deg_kernel()(dst2)
    hp1 = _tc1_call(x, W1, degp)
    acc1 = _get_agg_kernel()(hp1, src2, dst2)
    hp2 = _tc2_call(acc1, hp1, degp, b1.reshape(1, -1), g1.reshape(1, -1),
                    be1.reshape(1, -1), W2)
    acc2 = _get_agg_kernel()(hp2, src2, dst2)
    return _tc3_call(acc2, hp2, degp, b2.reshape(1, -1), g2.reshape(1, -1),
                     be2.reshape(1, -1), Wf1, bf1.reshape(1, -1),
                     Wf2, bf2.reshape(1, -1))
